# Initial kernel scaffold; baseline (speedup 1.0000x reference)
import jax
import jax.numpy as jnp
from jax.experimental import pallas as pl


def kernel(X, table):
    raise NotImplementedError("write your pallas kernel here")


# speedup vs baseline: 2.5126x; 2.5126x over previous
import functools

import jax
import jax.numpy as jnp
from jax import lax
from jax.experimental import pallas as pl
from jax.experimental.pallas import tpu as pltpu
from jax.experimental.pallas import tpu_sc as plsc

DIM = 16
NUM_CORES = 2
NUM_SUBCORES = 16
NUM_WORKERS = NUM_CORES * NUM_SUBCORES
CHUNK = 3200


@functools.partial(jax.jit, static_argnums=(2,))
def _gather_rows(flat_idx, table, n):
    per_worker = n // NUM_WORKERS
    chunks = per_worker // CHUNK
    mesh = plsc.VectorSubcoreMesh(core_axis_name="c", subcore_axis_name="s")

    @functools.partial(
        pl.kernel,
        mesh=mesh,
        out_type=jax.ShapeDtypeStruct((n, DIM), jnp.float32),
        scratch_types=[
            pltpu.VMEM((CHUNK,), jnp.int32),
            pltpu.VMEM((CHUNK, DIM), jnp.float32),
            pltpu.SemaphoreType.DMA,
        ],
        compiler_params=pltpu.CompilerParams(use_tc_tiling_on_sc=False),
    )
    def body(idx_hbm, table_hbm, out_hbm, idx_v, rows_v, sem):
        wid = lax.axis_index("s") * NUM_CORES + lax.axis_index("c")
        base = wid * per_worker

        def step(i, carry):
            off = base + i * CHUNK
            pltpu.sync_copy(idx_hbm.at[pl.ds(off, CHUNK)], idx_v)
            pltpu.async_copy(table_hbm.at[idx_v], rows_v, sem).wait()
            pltpu.sync_copy(rows_v, out_hbm.at[pl.ds(off, CHUNK)])
            return carry

        lax.fori_loop(0, chunks, step, 0)

    return body(flat_idx, table)


def kernel(X, table):
    b, l = X.shape
    n = b * l
    out = _gather_rows(X.reshape(n), table, n)
    return out.reshape(b, l, DIM)


# speedup vs baseline: 2.5184x; 1.0023x over previous
import functools

import jax
import jax.numpy as jnp
from jax import lax
from jax.experimental import pallas as pl
from jax.experimental.pallas import tpu as pltpu
from jax.experimental.pallas import tpu_sc as plsc

DIM = 16
NUM_CORES = 2
NUM_SUBCORES = 16
NUM_WORKERS = NUM_CORES * NUM_SUBCORES
CHUNK = 1600
NBUF = 2


@functools.partial(jax.jit, static_argnums=(2,))
def _gather_rows(flat_idx, table, n):
    per_worker = n // NUM_WORKERS
    chunks = per_worker // CHUNK
    groups = chunks // NBUF
    mesh = plsc.VectorSubcoreMesh(core_axis_name="c", subcore_axis_name="s")

    @functools.partial(
        pl.kernel,
        mesh=mesh,
        out_type=jax.ShapeDtypeStruct((n, DIM), jnp.float32),
        scratch_types=[
            pltpu.VMEM((NBUF, CHUNK), jnp.int32),
            pltpu.VMEM((NBUF, CHUNK, DIM), jnp.float32),
            pltpu.SemaphoreType.DMA,
            pltpu.SemaphoreType.DMA,
            pltpu.SemaphoreType.DMA,
            pltpu.SemaphoreType.DMA,
            pltpu.SemaphoreType.DMA,
            pltpu.SemaphoreType.DMA,
        ],
        compiler_params=pltpu.CompilerParams(use_tc_tiling_on_sc=False),
    )
    def body(idx_hbm, table_hbm, out_hbm, idx_v, rows_v,
             si0, si1, sg0, sg1, so0, so1):
        si = (si0, si1)
        sg = (sg0, sg1)
        so = (so0, so1)
        wid = lax.axis_index("s") * NUM_CORES + lax.axis_index("c")
        base = wid * per_worker

        def idx_copy(i, b):
            return pltpu.make_async_copy(
                idx_hbm.at[pl.ds(base + i * CHUNK, CHUNK)], idx_v.at[b], si[b])

        def gather_copy(b):
            return pltpu.make_async_copy(
                table_hbm.at[idx_v.at[b]], rows_v.at[b], sg[b])

        def out_copy(i, b):
            return pltpu.make_async_copy(
                rows_v.at[b], out_hbm.at[pl.ds(base + i * CHUNK, CHUNK)], so[b])

        for b in range(NBUF):
            idx_copy(b, b).start()

        def step(g, carry):
            for b in range(NBUF):
                i = g * NBUF + b
                idx_copy(i, b).wait()
                pl.when(g > 0)(lambda: out_copy(i - NBUF, b).wait())
                gather_copy(b).start()
                gather_copy(b).wait()
                out_copy(i, b).start()
                pl.when(i + NBUF < chunks)(
                    lambda: idx_copy(i + NBUF, b).start())
            return carry

        lax.fori_loop(0, groups, step, 0)
        for b in range(NBUF):
            out_copy(chunks - NBUF + b, b).wait()

    return body(flat_idx, table)


def kernel(X, table):
    b, l = X.shape
    n = b * l
    out = _gather_rows(X.reshape(n), table, n)
    return out.reshape(b, l, DIM)


# speedup vs baseline: 2.5204x; 1.0008x over previous
import functools

import jax
import jax.numpy as jnp
from jax import lax
from jax.experimental import pallas as pl
from jax.experimental.pallas import tpu as pltpu
from jax.experimental.pallas import tpu_sc as plsc

DIM = 16
NUM_CORES = 2
NUM_SUBCORES = 16
NUM_WORKERS = NUM_CORES * NUM_SUBCORES
SUB = 200
K = 8
CHUNK = SUB * K
NBUF = 2


@functools.partial(jax.jit, static_argnums=(2,))
def _gather_rows(idx2d, table, n):
    per_worker = n // NUM_WORKERS
    chunks = per_worker // CHUNK
    groups = chunks // NBUF
    mesh = plsc.VectorSubcoreMesh(core_axis_name="c", subcore_axis_name="s")

    @functools.partial(
        pl.kernel,
        mesh=mesh,
        out_type=jax.ShapeDtypeStruct((n // SUB, SUB, DIM), jnp.float32),
        scratch_types=[
            pltpu.VMEM((NBUF, K, SUB), jnp.int32),
            pltpu.VMEM((NBUF, K, SUB, DIM), jnp.float32),
            pltpu.SemaphoreType.DMA,
            pltpu.SemaphoreType.DMA,
            pltpu.SemaphoreType.DMA,
            pltpu.SemaphoreType.DMA,
            pltpu.SemaphoreType.DMA,
            pltpu.SemaphoreType.DMA,
        ],
        compiler_params=pltpu.CompilerParams(use_tc_tiling_on_sc=False),
    )
    def body(idx_hbm, table_hbm, out_hbm, idx_v, rows_v,
             si0, si1, sg0, sg1, so0, so1):
        si = (si0, si1)
        sg = (sg0, sg1)
        so = (so0, so1)
        wid = lax.axis_index("s") * NUM_CORES + lax.axis_index("c")
        base = wid * (per_worker // SUB)

        def idx_copy(i, b):
            return pltpu.make_async_copy(
                idx_hbm.at[pl.ds(base + i * K, K)], idx_v.at[b], si[b])

        def gather_copy(b, j):
            return pltpu.make_async_copy(
                table_hbm.at[idx_v.at[b, j]], rows_v.at[b, j], sg[b])

        def out_copy(i, b):
            return pltpu.make_async_copy(
                rows_v.at[b], out_hbm.at[pl.ds(base + i * K, K)], so[b])

        for b in range(NBUF):
            idx_copy(b, b).start()

        def step(g, carry):
            for b in range(NBUF):
                i = g * NBUF + b
                idx_copy(i, b).wait()
                pl.when(g > 0)(lambda: out_copy(i - NBUF, b).wait())
                for j in range(K):
                    gather_copy(b, j).start()
                for j in range(K):
                    gather_copy(b, j).wait()
                out_copy(i, b).start()
                pl.when(i + NBUF < chunks)(
                    lambda: idx_copy(i + NBUF, b).start())
            return carry

        lax.fori_loop(0, groups, step, 0)
        for b in range(NBUF):
            out_copy(chunks - NBUF + b, b).wait()

    return body(idx2d, table)


def kernel(X, table):
    b, l = X.shape
    n = b * l
    out = _gather_rows(X.reshape(n // SUB, SUB), table, n)
    return out.reshape(b, l, DIM)


# speedup vs baseline: 4.2544x; 1.6880x over previous
import functools

import jax
import jax.numpy as jnp
from jax import lax
from jax.experimental import pallas as pl
from jax.experimental.pallas import tpu as pltpu
from jax.experimental.pallas import tpu_sc as plsc

DIM = 16
NUM_CORES = 2
NUM_SUBCORES = 16
NUM_WORKERS = NUM_CORES * NUM_SUBCORES
CB = 4
CHUNK = CB * 128
SLAB = CB * 8 * 128


@functools.partial(jax.jit, static_argnums=(2, 3))
def _gather_rows(flat_idx_lmajor, table, bsz, lsz):
    nbc = bsz // 128
    mesh = plsc.VectorSubcoreMesh(core_axis_name="c", subcore_axis_name="s")

    @functools.partial(
        pl.kernel,
        mesh=mesh,
        out_type=jax.ShapeDtypeStruct((lsz, 2, nbc * 8 * 128), jnp.float32),
        scratch_types=[
            pltpu.VMEM((2, CHUNK), jnp.int32),
            pltpu.VMEM((2, CHUNK, DIM), jnp.float32),
            pltpu.VMEM((2 * 2 * SLAB,), jnp.float32),
            pltpu.SemaphoreType.DMA,
            pltpu.SemaphoreType.DMA,
            pltpu.SemaphoreType.DMA,
            pltpu.SemaphoreType.DMA,
            pltpu.SemaphoreType.DMA,
            pltpu.SemaphoreType.DMA,
        ],
        compiler_params=pltpu.CompilerParams(
            use_tc_tiling_on_sc=False, needs_layout_passes=False),
    )
    def body(idx_hbm, table_hbm, out_hbm, idx_v, rows_v, tp_v,
             si0, si1, sg0, sg1, so0, so1):
        si = (si0, si1)
        sg = (sg0, sg1)
        so = (so0, so1)
        wid = lax.axis_index("s") * NUM_CORES + lax.axis_index("c")
        col0 = wid * CB
        iota = lax.iota(jnp.int32, 16)
        lane_off = (iota // 8) * SLAB + (iota % 8) * 128

        def idx_copy(l, b):
            return pltpu.make_async_copy(
                idx_hbm.at[pl.ds(l * bsz + col0 * 128, CHUNK)],
                idx_v.at[b], si[b])

        def gather(b):
            return pltpu.make_async_copy(
                table_hbm.at[idx_v.at[b]], rows_v.at[b], sg[b])

        def store(l, b, dt):
            return pltpu.make_async_copy(
                tp_v.at[pl.ds((b * 2 + dt) * SLAB, SLAB)],
                out_hbm.at[l, dt, pl.ds(col0 * 8 * 128, SLAB)], so[b])

        def transpose(b):
            base = lane_off + b * (2 * SLAB)

            def per_row(r, carry):
                v = rows_v[b, r, :]
                pos = base + (r // 128) * 1024 + (r % 128)
                plsc.store_scatter(tp_v, [pos], v)
                return carry
            lax.fori_loop(0, CHUNK, per_row, 0)

        idx_copy(0, 0).start()
        idx_copy(0, 0).wait()
        gather(0).start()
        idx_copy(1, 1).start()

        def step(g, carry):
            for b in range(2):
                l = g * 2 + b
                b2 = 1 - b
                def launch_next():
                    idx_copy(l + 1, b2).wait()
                    gather(b2).start()
                pl.when(l + 1 < lsz)(launch_next)
                gather(b).wait()
                pl.when(l + 2 < lsz)(lambda: idx_copy(l + 2, b).start())
                def drain_prev():
                    store(l, b, 0).wait()
                    store(l, b, 1).wait()
                pl.when(l >= 2)(drain_prev)
                transpose(b)
                store(l, b, 0).start()
                store(l, b, 1).start()
            return carry

        lax.fori_loop(0, lsz // 2, step, 0)
        for b in range(2):
            store(0, b, 0).wait()
            store(0, b, 1).wait()

    return body(flat_idx_lmajor, table)


def kernel(X, table):
    b, l = X.shape
    p = _gather_rows(X.T.reshape(b * l), table, b, l)
    return (p.reshape(l, 2, b // 128, 8, 128)
            .transpose(2, 4, 0, 1, 3).reshape(b, l, DIM))


# speedup vs baseline: 4.9304x; 1.1589x over previous
import functools

import jax
import jax.numpy as jnp
from jax import lax
from jax.experimental import pallas as pl
from jax.experimental.pallas import tpu as pltpu
from jax.experimental.pallas import tpu_sc as plsc

DIM = 16
NUM_CORES = 2
NUM_SUBCORES = 16
NUM_WORKERS = NUM_CORES * NUM_SUBCORES
CB = 4
CHUNK = CB * 128
SLAB = CB * 8 * 128


@functools.partial(jax.jit, static_argnums=(2, 3))
def _gather_rows(flat_idx_lmajor, table, bsz, lsz):
    nbc = bsz // 128
    mesh = plsc.VectorSubcoreMesh(core_axis_name="c", subcore_axis_name="s")

    @functools.partial(
        pl.kernel,
        mesh=mesh,
        out_type=jax.ShapeDtypeStruct((lsz, 2, nbc * 8 * 128), jnp.float32),
        scratch_types=[
            pltpu.VMEM((2, CHUNK), jnp.int32),
            pltpu.VMEM((2, CHUNK, DIM), jnp.float32),
            pltpu.VMEM((2 * 2 * SLAB,), jnp.float32),
            pltpu.SemaphoreType.DMA,
            pltpu.SemaphoreType.DMA,
            pltpu.SemaphoreType.DMA,
            pltpu.SemaphoreType.DMA,
            pltpu.SemaphoreType.DMA,
            pltpu.SemaphoreType.DMA,
        ],
        compiler_params=pltpu.CompilerParams(
            use_tc_tiling_on_sc=False, needs_layout_passes=False),
    )
    def body(idx_hbm, table_hbm, out_hbm, idx_v, rows_v, tp_v,
             si0, si1, sg0, sg1, so0, so1):
        si = (si0, si1)
        sg = (sg0, sg1)
        so = (so0, so1)
        wid = lax.axis_index("s") * NUM_CORES + lax.axis_index("c")
        col0 = wid * CB
        iota = lax.iota(jnp.int32, 16)
        lane_off = (iota // 8) * SLAB + (iota % 8) * 128

        def idx_copy(l, b):
            return pltpu.make_async_copy(
                idx_hbm.at[pl.ds(l * bsz + col0 * 128, CHUNK)],
                idx_v.at[b], si[b])

        def gather(b):
            return pltpu.make_async_copy(
                table_hbm.at[idx_v.at[b]], rows_v.at[b], sg[b])

        def store(l, b, dt):
            return pltpu.make_async_copy(
                tp_v.at[pl.ds((b * 2 + dt) * SLAB, SLAB)],
                out_hbm.at[l, dt, pl.ds(col0 * 8 * 128, SLAB)], so[b])

        giota = [g * 16 + iota for g in range(8)]
        dcol = [jnp.full((16,), d, jnp.int32) for d in range(DIM)]
        bsel = [jnp.full((16,), b, jnp.int32) for b in range(2)]

        def transpose(b):
            def per_bc(bc, carry):
                rb = bc * 128
                tb = b * (2 * SLAB) + bc * 1024
                for d in range(DIM):
                    dt, d8 = divmod(d, 8)
                    for g in range(8):
                        v = plsc.load_gather(
                            rows_v, [bsel[b], rb + giota[g], dcol[d]])
                        tp_v[pl.ds(tb + dt * SLAB + d8 * 128 + g * 16, 16)] = v
                return carry
            lax.fori_loop(0, CB, per_bc, 0)

        idx_copy(0, 0).start()
        idx_copy(0, 0).wait()
        gather(0).start()
        idx_copy(1, 1).start()

        def step(g, carry):
            for b in range(2):
                l = g * 2 + b
                b2 = 1 - b
                def launch_next():
                    idx_copy(l + 1, b2).wait()
                    gather(b2).start()
                pl.when(l + 1 < lsz)(launch_next)
                gather(b).wait()
                pl.when(l + 2 < lsz)(lambda: idx_copy(l + 2, b).start())
                def drain_prev():
                    store(l, b, 0).wait()
                    store(l, b, 1).wait()
                pl.when(l >= 2)(drain_prev)
                transpose(b)
                store(l, b, 0).start()
                store(l, b, 1).start()
            return carry

        lax.fori_loop(0, lsz // 2, step, 0)
        for b in range(2):
            store(0, b, 0).wait()
            store(0, b, 1).wait()

    return body(flat_idx_lmajor, table)


def kernel(X, table):
    b, l = X.shape
    p = _gather_rows(X.T.reshape(b * l), table, b, l)
    return (p.reshape(l, 2, b // 128, 8, 128)
            .transpose(2, 4, 0, 1, 3).reshape(b, l, DIM))


# speedup vs baseline: 8.3094x; 1.6853x over previous
import functools

import jax
import jax.numpy as jnp
from jax import lax
from jax.experimental import pallas as pl
from jax.experimental.pallas import tpu as pltpu
from jax.experimental.pallas import tpu_sc as plsc

DIM = 16
NUM_CORES = 2
NUM_SUBCORES = 16
NUM_WORKERS = NUM_CORES * NUM_SUBCORES
CB = 4
CHUNK = CB * 128
SLAB = CB * 8 * 128


@functools.partial(jax.jit, static_argnums=(2, 3))
def _gather_rows(flat_idx_lmajor, table, bsz, lsz):
    nbc = bsz // 128
    mesh = plsc.VectorSubcoreMesh(core_axis_name="c", subcore_axis_name="s")

    @functools.partial(
        pl.kernel,
        mesh=mesh,
        out_type=jax.ShapeDtypeStruct((lsz, 2, nbc * 8 * 128), jnp.float32),
        scratch_types=[
            pltpu.VMEM((2, CHUNK), jnp.int32),
            pltpu.VMEM((2, CHUNK, DIM), jnp.float32),
            pltpu.VMEM((2 * 2 * SLAB,), jnp.float32),
            pltpu.SemaphoreType.DMA,
            pltpu.SemaphoreType.DMA,
            pltpu.SemaphoreType.DMA,
            pltpu.SemaphoreType.DMA,
            pltpu.SemaphoreType.DMA,
            pltpu.SemaphoreType.DMA,
        ],
        compiler_params=pltpu.CompilerParams(
            use_tc_tiling_on_sc=False, needs_layout_passes=False),
    )
    def body(idx_hbm, table_hbm, out_hbm, idx_v, rows_v, tp_v,
             si0, si1, sg0, sg1, so0, so1):
        si = (si0, si1)
        sg = (sg0, sg1)
        so = (so0, so1)
        wid = lax.axis_index("s") * NUM_CORES + lax.axis_index("c")
        col0 = wid * CB
        iota = lax.iota(jnp.int32, 16)
        lane_off = (iota // 8) * SLAB + (iota % 8) * 128

        def idx_copy(l, b):
            return pltpu.make_async_copy(
                idx_hbm.at[pl.ds(l * bsz + col0 * 128, CHUNK)],
                idx_v.at[b], si[b])

        def gather(b):
            return pltpu.make_async_copy(
                table_hbm.at[idx_v.at[b]], rows_v.at[b], sg[b])

        def store(l, b, dt):
            return pltpu.make_async_copy(
                tp_v.at[pl.ds((b * 2 + dt) * SLAB, SLAB)],
                out_hbm.at[l, dt, pl.ds(col0 * 8 * 128, SLAB)], so[b])

        giota = [g * 16 + iota for g in range(8)]
        dcol = [jnp.full((16,), d, jnp.int32) for d in range(DIM)]
        bsel = [jnp.full((16,), b, jnp.int32) for b in range(2)]

        def transpose(b):
            def per_bc(bc, carry):
                rb = bc * 128
                tb = b * (2 * SLAB) + bc * 1024
                for g in range(8):
                    row = rb + giota[g]
                    vs = [plsc.load_gather(rows_v, [bsel[b], row, dcol[d]])
                          for d in range(DIM)]
                    for d in range(DIM):
                        dt, d8 = divmod(d, 8)
                        tp_v[pl.ds(tb + dt * SLAB + d8 * 128 + g * 16, 16)] = (
                            vs[d])
                return carry
            lax.fori_loop(0, CB, per_bc, 0)

        idx_copy(0, 0).start()
        idx_copy(0, 0).wait()
        gather(0).start()
        idx_copy(1, 1).start()

        def step(g, carry):
            for b in range(2):
                l = g * 2 + b
                b2 = 1 - b
                def launch_next():
                    idx_copy(l + 1, b2).wait()
                    gather(b2).start()
                pl.when(l + 1 < lsz)(launch_next)
                gather(b).wait()
                pl.when(l + 2 < lsz)(lambda: idx_copy(l + 2, b).start())
                def drain_prev():
                    store(l, b, 0).wait()
                    store(l, b, 1).wait()
                pl.when(l >= 2)(drain_prev)
                transpose(b)
                store(l, b, 0).start()
                store(l, b, 1).start()
            return carry

        lax.fori_loop(0, lsz // 2, step, 0)
        for b in range(2):
            store(0, b, 0).wait()
            store(0, b, 1).wait()

    return body(flat_idx_lmajor, table)


def kernel(X, table):
    b, l = X.shape
    p = _gather_rows(X.T.reshape(b * l), table, b, l)
    return (p.reshape(l, 2, b // 128, 8, 128)
            .transpose(2, 4, 0, 1, 3).reshape(b, l, DIM))


# speedup vs baseline: 8.7010x; 1.0471x over previous
import functools

import jax
import jax.numpy as jnp
from jax import lax
from jax.experimental import pallas as pl
from jax.experimental.pallas import tpu as pltpu
from jax.experimental.pallas import tpu_sc as plsc

DIM = 16
NUM_CORES = 2
NUM_SUBCORES = 16
NUM_WORKERS = NUM_CORES * NUM_SUBCORES
CB = 4
CHUNK = 1024


@functools.partial(jax.jit, static_argnums=(2, 3))
def _gather_rows(idx_slabs, table, bsz, lsz):
    nbc = bsz // 128
    nlt = lsz // 8
    chunks = nlt * CB
    mesh = plsc.VectorSubcoreMesh(core_axis_name="c", subcore_axis_name="s")

    @functools.partial(
        pl.kernel,
        mesh=mesh,
        out_type=jax.ShapeDtypeStruct((nlt, 8, 2, nbc, 1024), jnp.float32),
        scratch_types=[
            pltpu.VMEM((2, CHUNK), jnp.int32),
            pltpu.VMEM((2, CHUNK, DIM), jnp.float32),
            pltpu.VMEM((2, 8, 2, 1024), jnp.float32),
            pltpu.SemaphoreType.DMA,
            pltpu.SemaphoreType.DMA,
            pltpu.SemaphoreType.DMA,
            pltpu.SemaphoreType.DMA,
            pltpu.SemaphoreType.DMA,
            pltpu.SemaphoreType.DMA,
        ],
        compiler_params=pltpu.CompilerParams(
            use_tc_tiling_on_sc=False, needs_layout_passes=False),
    )
    def body(idx_hbm, table_hbm, out_hbm, idx_v, rows_v, tp_v,
             si0, si1, sg0, sg1, so0, so1):
        si = (si0, si1)
        sg = (sg0, sg1)
        so = (so0, so1)
        wid = lax.axis_index("s") * NUM_CORES + lax.axis_index("c")
        col0 = wid * CB
        iota = lax.iota(jnp.int32, 16)
        giota = [g * 16 + iota for g in range(8)]
        dcol = [jnp.full((16,), d, jnp.int32) for d in range(DIM)]
        bsel = [jnp.full((16,), b, jnp.int32) for b in range(2)]

        def idx_copy(c, b):
            return pltpu.make_async_copy(
                idx_hbm.at[c // CB, col0 + c % CB], idx_v.at[b], si[b])

        def gather(b):
            return pltpu.make_async_copy(
                table_hbm.at[idx_v.at[b]], rows_v.at[b], sg[b])

        def store_one(c, b, l8, dt):
            return pltpu.make_async_copy(
                tp_v.at[b, l8, dt],
                out_hbm.at[c // CB, l8, dt, col0 + c % CB], so[b])

        def stores_start(c, b):
            for l8 in range(8):
                for dt in range(2):
                    store_one(c, b, l8, dt).start()

        def stores_wait(c, b):
            for l8 in range(8):
                for dt in range(2):
                    store_one(c, b, l8, dt).wait()

        def transpose(b):
            def per_l8(l8, carry):
                rb = l8 * 128
                for g in range(8):
                    row = rb + giota[g]
                    vs = [plsc.load_gather(rows_v, [bsel[b], row, dcol[d]])
                          for d in range(DIM)]
                    for d in range(DIM):
                        dt, d8 = divmod(d, 8)
                        tp_v[b, l8, dt, pl.ds(d8 * 128 + g * 16, 16)] = vs[d]
                return carry
            lax.fori_loop(0, 8, per_l8, 0)

        idx_copy(0, 0).start()
        idx_copy(0, 0).wait()
        gather(0).start()
        idx_copy(1, 1).start()

        def step(g, carry):
            for b in range(2):
                c = g * 2 + b
                b2 = 1 - b
                def launch_next():
                    idx_copy(c + 1, b2).wait()
                    gather(b2).start()
                pl.when(c + 1 < chunks)(launch_next)
                gather(b).wait()
                pl.when(c + 2 < chunks)(lambda: idx_copy(c + 2, b).start())
                pl.when(c >= 2)(lambda: stores_wait(c, b))
                transpose(b)
                stores_start(c, b)
            return carry

        lax.fori_loop(0, chunks // 2, step, 0)
        for b in range(2):
            stores_wait(0, b)

    return body(idx_slabs, table)


def kernel(X, table):
    b, l = X.shape
    idx_slabs = (X.reshape(b // 128, 128, l // 8, 8)
                 .transpose(2, 0, 3, 1).reshape(l // 8, 128, 1024))
    p = _gather_rows(idx_slabs, table, b, l)
    return (p.reshape(l, 2, b // 128, 8, 128)
            .transpose(2, 4, 0, 1, 3).reshape(b, l, DIM))
